# baseline (device time: 64491 ns/iter reference)
import jax
import jax.numpy as jnp
from jax import lax
from jax.experimental import pallas as pl
from jax.experimental.pallas import tpu as pltpu

N_DEV = 16
LOG2_N = 4
B, SQ, D = 2, 128, 512
HQ_LOC, DH = 8, 64
SKV = 128
T = B * SQ
NC = 2
CR = T // NC


def kernel(x, Wq, Wo, K_ext, V_ext):
    my_i = lax.axis_index("i")

    x2d = x.reshape(T, D)
    k2 = K_ext.reshape(B * SKV, 128 * DH)
    v2 = V_ext.reshape(B * SKV, 128 * DH)

    def body(x_ref, wq_ref, wo_ref, k_hbm, v_hbm, out_ref,
             acc_ref, o_ref, recv_ref, xb, wqb, wob, kf, vf,
             send_sems, recv_sems, k_dma_sems, v_dma_sems):
        my = lax.axis_index("i")

        kv_copies = []
        for b in range(B):
            for p in range(HQ_LOC // 2):
                cols = pl.ds((my * HQ_LOC + 2 * p) * DH, 2 * DH)
                rows = pl.ds(b * SKV, SKV)
                ck = pltpu.make_async_copy(
                    k_hbm.at[rows, cols], kf.at[b, p], k_dma_sems.at[b, p]
                )
                cv = pltpu.make_async_copy(
                    v_hbm.at[rows, cols], vf.at[b, p], v_dma_sems.at[b, p]
                )
                ck.start()
                cv.start()
                kv_copies.append((ck, cv))

        barrier = pltpu.get_barrier_semaphore()
        for step in range(LOG2_N):
            partner = my ^ (1 << step)
            pl.semaphore_signal(
                barrier, inc=1,
                device_id=(partner,), device_id_type=pl.DeviceIdType.MESH,
            )

        xb[...] = x_ref[...].astype(jnp.bfloat16)
        wqb[...] = wq_ref[...].astype(jnp.bfloat16)
        wob[...] = wo_ref[...].astype(jnp.bfloat16)

        q2d = jnp.dot(
            xb[...], wqb[...], preferred_element_type=jnp.float32
        ).astype(jnp.bfloat16)

        for ck, cv in kv_copies:
            ck.wait()
            cv.wait()

        def attend(b):
            for h in range(HQ_LOC):
                p_i, half = divmod(h, 2)
                q = q2d[b * SQ:(b + 1) * SQ, h * DH:(h + 1) * DH]
                kk = kf[b, p_i][:, half * DH:(half + 1) * DH].astype(
                    jnp.bfloat16
                )
                vv = vf[b, p_i][:, half * DH:(half + 1) * DH].astype(
                    jnp.bfloat16
                )
                s = lax.dot_general(
                    q, kk, (((1,), (1,)), ((), ())),
                    preferred_element_type=jnp.float32,
                ) * 0.125
                m = jnp.max(s, axis=-1, keepdims=True)
                p = jnp.exp(s - m)
                l = jnp.sum(p, axis=-1, keepdims=True)
                o = lax.dot_general(
                    p.astype(jnp.bfloat16), vv, (((1,), (0,)), ((), ())),
                    preferred_element_type=jnp.float32,
                )
                o_ref[b * SQ:(b + 1) * SQ, h * DH:(h + 1) * DH] = (
                    o / l
                ).astype(jnp.bfloat16)
            acc_ref[b, :, :] = jnp.dot(
                o_ref[pl.ds(b * CR, CR), :], wob[...],
                preferred_element_type=jnp.float32,
            ).astype(jnp.bfloat16)

        rdmas = {}

        def issue(step, c):
            partner = my ^ (1 << step)
            r = pltpu.make_async_remote_copy(
                src_ref=acc_ref.at[c],
                dst_ref=recv_ref.at[step, c],
                send_sem=send_sems.at[step, c],
                recv_sem=recv_sems.at[step, c],
                device_id=(partner,),
                device_id_type=pl.DeviceIdType.MESH,
            )
            r.start()
            rdmas[(step, c)] = r

        def finish(step, c):
            rdmas[(step, c)].wait()
            acc_ref[c, :, :] = (
                acc_ref[c].astype(jnp.float32)
                + recv_ref[step, c].astype(jnp.float32)
            ).astype(jnp.bfloat16)

        attend(0)
        pl.semaphore_wait(barrier, LOG2_N)
        issue(0, 0)
        attend(1)
        issue(0, 1)
        for step in range(LOG2_N):
            finish(step, 0)
            if step + 1 < LOG2_N:
                issue(step + 1, 0)
            finish(step, 1)
            if step + 1 < LOG2_N:
                issue(step + 1, 1)

        out_ref[...] = acc_ref[...].reshape(B, SQ, D).astype(jnp.float32)

    return pl.pallas_call(
        body,
        out_shape=jax.ShapeDtypeStruct((B, SQ, D), jnp.float32),
        in_specs=[
            pl.BlockSpec(memory_space=pltpu.VMEM),
            pl.BlockSpec(memory_space=pltpu.VMEM),
            pl.BlockSpec(memory_space=pltpu.VMEM),
            pl.BlockSpec(memory_space=pltpu.MemorySpace.HBM),
            pl.BlockSpec(memory_space=pltpu.MemorySpace.HBM),
        ],
        out_specs=pl.BlockSpec(memory_space=pltpu.VMEM),
        scratch_shapes=[
            pltpu.VMEM((NC, CR, D), jnp.bfloat16),
            pltpu.VMEM((T, D), jnp.bfloat16),
            pltpu.VMEM((LOG2_N, NC, CR, D), jnp.bfloat16),
            pltpu.VMEM((T, D), jnp.bfloat16),
            pltpu.VMEM((D, D), jnp.bfloat16),
            pltpu.VMEM((D, D), jnp.bfloat16),
            pltpu.VMEM((B, HQ_LOC // 2, SKV, 2 * DH), jnp.float32),
            pltpu.VMEM((B, HQ_LOC // 2, SKV, 2 * DH), jnp.float32),
            pltpu.SemaphoreType.DMA((LOG2_N, NC)),
            pltpu.SemaphoreType.DMA((LOG2_N, NC)),
            pltpu.SemaphoreType.DMA((B, HQ_LOC // 2)),
            pltpu.SemaphoreType.DMA((B, HQ_LOC // 2)),
        ],
        compiler_params=pltpu.CompilerParams(collective_id=0),
    )(x2d, Wq, Wo, k2, v2)


# device time: 41699 ns/iter; 1.5466x vs baseline; 1.5466x over previous
import jax
import jax.numpy as jnp
from jax import lax
from jax.experimental import pallas as pl
from jax.experimental.pallas import tpu as pltpu

N_DEV = 16
LOG2_N = 4
B, SQ, D = 2, 128, 512
HQ_LOC, DH = 8, 64
SKV = 128
T = B * SQ
NC = 2
CR = T // NC


def kernel(x, Wq, Wo, K_ext, V_ext):
    my_i = lax.axis_index("i")

    k_sl = lax.dynamic_slice_in_dim(K_ext, my_i * HQ_LOC, HQ_LOC, axis=2)
    v_sl = lax.dynamic_slice_in_dim(V_ext, my_i * HQ_LOC, HQ_LOC, axis=2)
    k_sl = k_sl.astype(jnp.bfloat16).reshape(B * SKV, HQ_LOC * DH)
    v_sl = v_sl.astype(jnp.bfloat16).reshape(B * SKV, HQ_LOC * DH)
    x2d = x.reshape(T, D).astype(jnp.bfloat16)
    wq = Wq.astype(jnp.bfloat16)
    wo = Wo.astype(jnp.bfloat16)

    def body(x_ref, wq_ref, wo_ref, k_ref, v_ref, out_ref,
             acc_ref, o_ref, recv_ref, send_sems, recv_sems):
        my = lax.axis_index("i")

        barrier = pltpu.get_barrier_semaphore()
        for step in range(LOG2_N):
            partner = my ^ (1 << step)
            pl.semaphore_signal(
                barrier, inc=1,
                device_id=(partner,), device_id_type=pl.DeviceIdType.MESH,
            )

        q2d = jnp.dot(
            x_ref[...], wq_ref[...], preferred_element_type=jnp.float32
        ).astype(jnp.bfloat16)

        lane = lax.broadcasted_iota(jnp.int32, (1, 2 * DH), 1)
        m0 = (lane < DH).astype(jnp.bfloat16)
        m1 = (lane >= DH).astype(jnp.bfloat16)
        m0f = m0.astype(jnp.float32)
        m1f = m1.astype(jnp.float32)

        def attend(b):
            for p_i in range(HQ_LOC // 2):
                rows = slice(b * SQ, (b + 1) * SQ)
                cols = slice(p_i * 2 * DH, (p_i + 1) * 2 * DH)
                qp = q2d[rows, cols]
                kp = k_ref[rows, cols]
                vp = v_ref[rows, cols]
                dn_t = (((1,), (1,)), ((), ()))
                dn_n = (((1,), (0,)), ((), ()))
                s0 = lax.dot_general(
                    qp, kp * m0, dn_t, preferred_element_type=jnp.float32
                ) * 0.125
                s1 = lax.dot_general(
                    qp, kp * m1, dn_t, preferred_element_type=jnp.float32
                ) * 0.125
                p0 = jnp.exp(s0 - jnp.max(s0, axis=-1, keepdims=True))
                p1 = jnp.exp(s1 - jnp.max(s1, axis=-1, keepdims=True))
                l0 = jnp.sum(p0, axis=-1, keepdims=True)
                l1 = jnp.sum(p1, axis=-1, keepdims=True)
                op = lax.dot_general(
                    p0.astype(jnp.bfloat16), vp * m0, dn_n,
                    preferred_element_type=jnp.float32,
                ) + lax.dot_general(
                    p1.astype(jnp.bfloat16), vp * m1, dn_n,
                    preferred_element_type=jnp.float32,
                )
                lp = l0 * m0f + l1 * m1f
                o_ref[rows, cols] = (op / lp).astype(jnp.bfloat16)
            acc_ref[b, :, :] = jnp.dot(
                o_ref[pl.ds(b * CR, CR), :], wo_ref[...],
                preferred_element_type=jnp.float32,
            ).astype(jnp.bfloat16)

        rdmas = {}

        def issue(step, c):
            partner = my ^ (1 << step)
            r = pltpu.make_async_remote_copy(
                src_ref=acc_ref.at[c],
                dst_ref=recv_ref.at[step, c],
                send_sem=send_sems.at[step, c],
                recv_sem=recv_sems.at[step, c],
                device_id=(partner,),
                device_id_type=pl.DeviceIdType.MESH,
            )
            r.start()
            rdmas[(step, c)] = r

        def finish(step, c):
            rdmas[(step, c)].wait()
            acc_ref[c, :, :] = (
                acc_ref[c].astype(jnp.float32)
                + recv_ref[step, c].astype(jnp.float32)
            ).astype(jnp.bfloat16)

        attend(0)
        pl.semaphore_wait(barrier, LOG2_N)
        issue(0, 0)
        attend(1)
        issue(0, 1)
        for step in range(LOG2_N):
            finish(step, 0)
            if step + 1 < LOG2_N:
                issue(step + 1, 0)
            finish(step, 1)
            if step + 1 < LOG2_N:
                issue(step + 1, 1)

        out_ref[...] = acc_ref[...].reshape(B, SQ, D).astype(jnp.float32)

    return pl.pallas_call(
        body,
        out_shape=jax.ShapeDtypeStruct((B, SQ, D), jnp.float32),
        in_specs=[pl.BlockSpec(memory_space=pltpu.VMEM)] * 5,
        out_specs=pl.BlockSpec(memory_space=pltpu.VMEM),
        scratch_shapes=[
            pltpu.VMEM((NC, CR, D), jnp.bfloat16),
            pltpu.VMEM((T, D), jnp.bfloat16),
            pltpu.VMEM((LOG2_N, NC, CR, D), jnp.bfloat16),
            pltpu.SemaphoreType.DMA((LOG2_N, NC)),
            pltpu.SemaphoreType.DMA((LOG2_N, NC)),
        ],
        compiler_params=pltpu.CompilerParams(collective_id=0),
    )(x2d, wq, wo, k_sl, v_sl)


# device time: 36678 ns/iter; 1.7583x vs baseline; 1.1369x over previous
import jax
import jax.numpy as jnp
from jax import lax
from jax.experimental import pallas as pl
from jax.experimental.pallas import tpu as pltpu

N_DEV = 16
LOG2_N = 4
B, SQ, D = 2, 128, 512
HQ_LOC, DH = 8, 64
SKV = 128
T = B * SQ
NC = 2
CR = T // NC


def kernel(x, Wq, Wo, K_ext, V_ext):
    my_i = lax.axis_index("i")

    k_sl = lax.dynamic_slice_in_dim(K_ext, my_i * HQ_LOC, HQ_LOC, axis=2)
    v_sl = lax.dynamic_slice_in_dim(V_ext, my_i * HQ_LOC, HQ_LOC, axis=2)
    k_sl = k_sl.transpose(0, 2, 1, 3).astype(jnp.bfloat16)
    v_sl = v_sl.transpose(0, 2, 1, 3).astype(jnp.bfloat16)
    x2d = x.reshape(T, D)

    def body(x_ref, wq_ref, wo_ref, k_ref, v_ref, out_ref,
             acc_ref, o_ref, recv_ref, send_sems, recv_sems):
        my = lax.axis_index("i")

        barrier = pltpu.get_barrier_semaphore()
        for step in range(LOG2_N):
            partner = my ^ (1 << step)
            pl.semaphore_signal(
                barrier, inc=1,
                device_id=(partner,), device_id_type=pl.DeviceIdType.MESH,
            )

        q2d = jnp.dot(
            x_ref[...], wq_ref[...], preferred_element_type=jnp.float32
        ).astype(jnp.bfloat16)

        def attend(b):
            for h in range(HQ_LOC):
                q = q2d[b * SQ:(b + 1) * SQ, h * DH:(h + 1) * DH]
                kk = k_ref[b, h]
                vv = v_ref[b, h]
                s = lax.dot_general(
                    q, kk, (((1,), (1,)), ((), ())),
                    preferred_element_type=jnp.float32,
                ) * 0.125
                m = jnp.max(s, axis=-1, keepdims=True)
                p = jnp.exp(s - m)
                l = jnp.sum(p, axis=-1, keepdims=True)
                o = lax.dot_general(
                    p.astype(jnp.bfloat16), vv, (((1,), (0,)), ((), ())),
                    preferred_element_type=jnp.float32,
                )
                o_ref[b * SQ:(b + 1) * SQ, h * DH:(h + 1) * DH] = o / l
            acc_ref[b, :, :] = jnp.dot(
                o_ref[pl.ds(b * CR, CR), :], wo_ref[...],
                preferred_element_type=jnp.float32,
            ).astype(jnp.bfloat16)

        rdmas = {}

        def issue(step, c):
            partner = my ^ (1 << step)
            r = pltpu.make_async_remote_copy(
                src_ref=acc_ref.at[c],
                dst_ref=recv_ref.at[step, c],
                send_sem=send_sems.at[step, c],
                recv_sem=recv_sems.at[step, c],
                device_id=(partner,),
                device_id_type=pl.DeviceIdType.MESH,
            )
            r.start()
            rdmas[(step, c)] = r

        def finish(step, c):
            rdmas[(step, c)].wait()
            acc_ref[c, :, :] = (
                acc_ref[c].astype(jnp.float32)
                + recv_ref[step, c].astype(jnp.float32)
            ).astype(jnp.bfloat16)

        attend(0)
        pl.semaphore_wait(barrier, LOG2_N)
        issue(0, 0)
        attend(1)
        issue(0, 1)
        for step in range(LOG2_N):
            finish(step, 0)
            if step + 1 < LOG2_N:
                issue(step + 1, 0)
            finish(step, 1)
            if step + 1 < LOG2_N:
                issue(step + 1, 1)

        out_ref[...] = acc_ref[...].reshape(B, SQ, D).astype(jnp.float32)

    return pl.pallas_call(
        body,
        out_shape=jax.ShapeDtypeStruct((B, SQ, D), jnp.float32),
        in_specs=[pl.BlockSpec(memory_space=pltpu.VMEM)] * 5,
        out_specs=pl.BlockSpec(memory_space=pltpu.VMEM),
        scratch_shapes=[
            pltpu.VMEM((NC, CR, D), jnp.bfloat16),
            pltpu.VMEM((T, D), jnp.float32),
            pltpu.VMEM((LOG2_N, NC, CR, D), jnp.bfloat16),
            pltpu.SemaphoreType.DMA((LOG2_N, NC)),
            pltpu.SemaphoreType.DMA((LOG2_N, NC)),
        ],
        compiler_params=pltpu.CompilerParams(collective_id=0),
    )(x2d, Wq, Wo, k_sl, v_sl)


# device time: 34105 ns/iter; 1.8910x vs baseline; 1.0754x over previous
import jax
import jax.numpy as jnp
from jax import lax
from jax.experimental import pallas as pl
from jax.experimental.pallas import tpu as pltpu

N_DEV = 16
LOG2_N = 4
PARTNER_XOR = (8, 4, 1, 3)
B, SQ, D = 2, 128, 512
HQ_LOC, DH = 8, 64
SKV = 128
T = B * SQ
NC = 2
CR = T // NC


def kernel(x, Wq, Wo, K_ext, V_ext):
    my_i = lax.axis_index("i")

    k_sl = lax.dynamic_slice_in_dim(K_ext, my_i * HQ_LOC, HQ_LOC, axis=2)
    v_sl = lax.dynamic_slice_in_dim(V_ext, my_i * HQ_LOC, HQ_LOC, axis=2)
    k_sl = k_sl.transpose(0, 2, 1, 3).astype(jnp.bfloat16)
    v_sl = v_sl.transpose(0, 2, 1, 3).astype(jnp.bfloat16)
    x2d = x.reshape(T, D)

    def body(x_ref, wq_ref, wo_ref, k_ref, v_ref, out_ref,
             acc_ref, o_ref, recv_ref, send_sems, recv_sems):
        my = lax.axis_index("i")

        barrier = pltpu.get_barrier_semaphore()
        for step in range(LOG2_N):
            partner = my ^ PARTNER_XOR[step]
            pl.semaphore_signal(
                barrier, inc=1,
                device_id=(partner,), device_id_type=pl.DeviceIdType.MESH,
            )

        q2d = jnp.dot(
            x_ref[...], wq_ref[...], preferred_element_type=jnp.float32
        ).astype(jnp.bfloat16)

        def attend(b):
            for h in range(HQ_LOC):
                q = q2d[b * SQ:(b + 1) * SQ, h * DH:(h + 1) * DH]
                kk = k_ref[b, h]
                vv = v_ref[b, h]
                s = lax.dot_general(
                    q, kk, (((1,), (1,)), ((), ())),
                    preferred_element_type=jnp.float32,
                ) * 0.125
                m = jnp.max(s, axis=-1, keepdims=True)
                p = jnp.exp(s - m)
                l = jnp.sum(p, axis=-1, keepdims=True)
                o = lax.dot_general(
                    p.astype(jnp.bfloat16), vv, (((1,), (0,)), ((), ())),
                    preferred_element_type=jnp.float32,
                )
                o_ref[b * SQ:(b + 1) * SQ, h * DH:(h + 1) * DH] = o / l
            acc_ref[b, :, :] = jnp.dot(
                o_ref[pl.ds(b * CR, CR), :], wo_ref[...],
                preferred_element_type=jnp.float32,
            ).astype(jnp.bfloat16)

        rdmas = {}

        def issue(step, c):
            partner = my ^ PARTNER_XOR[step]
            r = pltpu.make_async_remote_copy(
                src_ref=acc_ref.at[c],
                dst_ref=recv_ref.at[step, c],
                send_sem=send_sems.at[step, c],
                recv_sem=recv_sems.at[step, c],
                device_id=(partner,),
                device_id_type=pl.DeviceIdType.MESH,
            )
            r.start()
            rdmas[(step, c)] = r

        def finish(step, c):
            rdmas[(step, c)].wait()
            acc_ref[c, :, :] = (
                acc_ref[c].astype(jnp.float32)
                + recv_ref[step, c].astype(jnp.float32)
            ).astype(jnp.bfloat16)

        attend(0)
        pl.semaphore_wait(barrier, LOG2_N)
        issue(0, 0)
        attend(1)
        issue(0, 1)
        for step in range(LOG2_N):
            finish(step, 0)
            if step + 1 < LOG2_N:
                issue(step + 1, 0)
            else:
                out_ref[0, :, :] = acc_ref[0].astype(jnp.float32)
            finish(step, 1)
            if step + 1 < LOG2_N:
                issue(step + 1, 1)
        out_ref[1, :, :] = acc_ref[1].astype(jnp.float32)

    return pl.pallas_call(
        body,
        out_shape=jax.ShapeDtypeStruct((B, SQ, D), jnp.float32),
        in_specs=[pl.BlockSpec(memory_space=pltpu.VMEM)] * 5,
        out_specs=pl.BlockSpec(memory_space=pltpu.VMEM),
        scratch_shapes=[
            pltpu.VMEM((NC, CR, D), jnp.bfloat16),
            pltpu.VMEM((T, D), jnp.float32),
            pltpu.VMEM((LOG2_N, NC, CR, D), jnp.bfloat16),
            pltpu.SemaphoreType.DMA((LOG2_N, NC)),
            pltpu.SemaphoreType.DMA((LOG2_N, NC)),
        ],
        compiler_params=pltpu.CompilerParams(collective_id=0),
    )(x2d, Wq, Wo, k_sl, v_sl)


# device time: 34072 ns/iter; 1.8928x vs baseline; 1.0010x over previous
import jax
import jax.numpy as jnp
from jax import lax
from jax.experimental import pallas as pl
from jax.experimental.pallas import tpu as pltpu

N_DEV = 16
LOG2_N = 4
PARTNER_XOR = (8, 4, 1, 3)
B, SQ, D = 2, 128, 512
HQ_LOC, DH = 8, 64
SKV = 128
T = B * SQ
NC = 2
CR = T // NC


def kernel(x, Wq, Wo, K_ext, V_ext):
    my_i = lax.axis_index("i")

    k_sl = lax.dynamic_slice_in_dim(K_ext, my_i * HQ_LOC, HQ_LOC, axis=2)
    v_sl = lax.dynamic_slice_in_dim(V_ext, my_i * HQ_LOC, HQ_LOC, axis=2)
    k_sl = k_sl.transpose(0, 2, 1, 3).astype(jnp.bfloat16)
    v_sl = v_sl.transpose(0, 2, 1, 3).astype(jnp.bfloat16)

    def body(x_ref, wq_ref, wo_ref, k_ref, v_ref, out_ref,
             acc_ref, o_ref, recv_ref, send_sems, recv_sems):
        my = lax.axis_index("i")

        barrier = pltpu.get_barrier_semaphore()
        for step in range(LOG2_N):
            partner = my ^ PARTNER_XOR[step]
            pl.semaphore_signal(
                barrier, inc=1,
                device_id=(partner,), device_id_type=pl.DeviceIdType.MESH,
            )

        q2d = jnp.dot(
            x_ref[...].reshape(T, D), wq_ref[...],
            preferred_element_type=jnp.float32,
        ).astype(jnp.bfloat16)

        def attend(b):
            for h in range(HQ_LOC):
                q = q2d[b * SQ:(b + 1) * SQ, h * DH:(h + 1) * DH]
                kk = k_ref[b, h]
                vv = v_ref[b, h]
                s = lax.dot_general(
                    q, kk, (((1,), (1,)), ((), ())),
                    preferred_element_type=jnp.float32,
                ) * 0.125
                m = jnp.max(s, axis=-1, keepdims=True)
                p = jnp.exp(s - m)
                l = jnp.sum(p, axis=-1, keepdims=True)
                o = lax.dot_general(
                    p.astype(jnp.bfloat16), vv, (((1,), (0,)), ((), ())),
                    preferred_element_type=jnp.float32,
                )
                o_ref[b * SQ:(b + 1) * SQ, h * DH:(h + 1) * DH] = o / l
            acc_ref[b, :, :] = jnp.dot(
                o_ref[pl.ds(b * CR, CR), :], wo_ref[...],
                preferred_element_type=jnp.float32,
            ).astype(jnp.bfloat16)

        rdmas = {}

        def issue(step, c):
            partner = my ^ PARTNER_XOR[step]
            r = pltpu.make_async_remote_copy(
                src_ref=acc_ref.at[c],
                dst_ref=recv_ref.at[step, c],
                send_sem=send_sems.at[step, c],
                recv_sem=recv_sems.at[step, c],
                device_id=(partner,),
                device_id_type=pl.DeviceIdType.MESH,
            )
            r.start()
            rdmas[(step, c)] = r

        def finish(step, c):
            rdmas[(step, c)].wait()
            acc_ref[c, :, :] = (
                acc_ref[c].astype(jnp.float32)
                + recv_ref[step, c].astype(jnp.float32)
            ).astype(jnp.bfloat16)

        attend(0)
        pl.semaphore_wait(barrier, LOG2_N)
        issue(0, 0)
        attend(1)
        issue(0, 1)
        for step in range(LOG2_N):
            finish(step, 0)
            if step + 1 < LOG2_N:
                issue(step + 1, 0)
            else:
                out_ref[0, :, :] = acc_ref[0].astype(jnp.float32)
            finish(step, 1)
            if step + 1 < LOG2_N:
                issue(step + 1, 1)
        out_ref[1, :, :] = acc_ref[1].astype(jnp.float32)

    return pl.pallas_call(
        body,
        out_shape=jax.ShapeDtypeStruct((B, SQ, D), jnp.float32),
        in_specs=[pl.BlockSpec(memory_space=pltpu.VMEM)] * 5,
        out_specs=pl.BlockSpec(memory_space=pltpu.VMEM),
        scratch_shapes=[
            pltpu.VMEM((NC, CR, D), jnp.bfloat16),
            pltpu.VMEM((T, D), jnp.float32),
            pltpu.VMEM((LOG2_N, NC, CR, D), jnp.bfloat16),
            pltpu.SemaphoreType.DMA((LOG2_N, NC)),
            pltpu.SemaphoreType.DMA((LOG2_N, NC)),
        ],
        compiler_params=pltpu.CompilerParams(collective_id=0),
    )(x, Wq, Wo, k_sl, v_sl)


# device time: 33990 ns/iter; 1.8974x vs baseline; 1.0024x over previous
import jax
import jax.numpy as jnp
from jax import lax
from jax.experimental import pallas as pl
from jax.experimental.pallas import tpu as pltpu

N_DEV = 16
LOG2_N = 4
PARTNER_XOR = (8, 4, 1, 3)
B, SQ, D = 2, 128, 512
HQ_LOC, DH = 8, 64
SKV = 128
T = B * SQ
NC = 2
CR = T // NC


def kernel(x, Wq, Wo, K_ext, V_ext):
    my_i = lax.axis_index("i")

    k_sl = lax.dynamic_slice_in_dim(K_ext, my_i * HQ_LOC, HQ_LOC, axis=2)
    v_sl = lax.dynamic_slice_in_dim(V_ext, my_i * HQ_LOC, HQ_LOC, axis=2)
    k_sl = k_sl.transpose(0, 2, 1, 3).astype(jnp.bfloat16)
    v_sl = v_sl.transpose(0, 2, 1, 3).astype(jnp.bfloat16)

    def body(x_ref, wq_ref, wo_ref, k_ref, v_ref, out_ref,
             acc_ref, o_ref, recv_ref, send_sems, recv_sems):
        my = lax.axis_index("i")

        barrier = pltpu.get_barrier_semaphore()
        for step in range(LOG2_N):
            partner = my ^ PARTNER_XOR[step]
            pl.semaphore_signal(
                barrier, inc=1,
                device_id=(partner,), device_id_type=pl.DeviceIdType.MESH,
            )

        q2d = jnp.dot(
            x_ref[...].reshape(T, D), wq_ref[...],
            preferred_element_type=jnp.float32,
        ).astype(jnp.bfloat16)

        def attend(b):
            for h in range(HQ_LOC):
                q = q2d[b * SQ:(b + 1) * SQ, h * DH:(h + 1) * DH]
                kk = k_ref[b, h]
                vv = v_ref[b, h]
                s = lax.dot_general(
                    q, kk, (((1,), (1,)), ((), ())),
                    preferred_element_type=jnp.float32,
                ) * 0.125
                m = jnp.max(s, axis=-1, keepdims=True)
                p = jnp.exp(s - m)
                l = jnp.sum(p, axis=-1, keepdims=True)
                o = lax.dot_general(
                    p.astype(jnp.bfloat16), vv, (((1,), (0,)), ((), ())),
                    preferred_element_type=jnp.float32,
                )
                o_ref[b * SQ:(b + 1) * SQ, h * DH:(h + 1) * DH] = o / l
            acc_ref[b, :, :] = jnp.dot(
                o_ref[pl.ds(b * CR, CR), :], wo_ref[...],
                preferred_element_type=jnp.float32,
            ).astype(jnp.bfloat16)

        rdmas = {}

        def issue(step, c):
            partner = my ^ PARTNER_XOR[step]
            r = pltpu.make_async_remote_copy(
                src_ref=acc_ref.at[c],
                dst_ref=recv_ref.at[step, c],
                send_sem=send_sems.at[step, c],
                recv_sem=recv_sems.at[step, c],
                device_id=(partner,),
                device_id_type=pl.DeviceIdType.MESH,
            )
            r.start()
            rdmas[(step, c)] = r

        def finish(step, c):
            rdmas[(step, c)].wait()
            acc_ref[c, :, :] = acc_ref[c] + recv_ref[step, c]

        attend(0)
        pl.semaphore_wait(barrier, LOG2_N)
        issue(0, 0)
        attend(1)
        issue(0, 1)
        for step in range(LOG2_N):
            finish(step, 0)
            if step + 1 < LOG2_N:
                issue(step + 1, 0)
            else:
                out_ref[0, :, :] = acc_ref[0].astype(jnp.float32)
            finish(step, 1)
            if step + 1 < LOG2_N:
                issue(step + 1, 1)
        out_ref[1, :, :] = acc_ref[1].astype(jnp.float32)

    return pl.pallas_call(
        body,
        out_shape=jax.ShapeDtypeStruct((B, SQ, D), jnp.float32),
        in_specs=[pl.BlockSpec(memory_space=pltpu.VMEM)] * 5,
        out_specs=pl.BlockSpec(memory_space=pltpu.VMEM),
        scratch_shapes=[
            pltpu.VMEM((NC, CR, D), jnp.bfloat16),
            pltpu.VMEM((T, D), jnp.float32),
            pltpu.VMEM((LOG2_N, NC, CR, D), jnp.bfloat16),
            pltpu.SemaphoreType.DMA((LOG2_N, NC)),
            pltpu.SemaphoreType.DMA((LOG2_N, NC)),
        ],
        compiler_params=pltpu.CompilerParams(collective_id=0),
    )(x, Wq, Wo, k_sl, v_sl)
